# baseline (speedup 1.0000x reference)
import functools

import jax
import jax.numpy as jnp
from jax import lax
from jax.experimental import pallas as pl
from jax.experimental.pallas import tpu as pltpu
from jax.experimental.pallas import tpu_sc as plsc

_OUTC = 32
_R1 = 64
_R2 = 128
_NPTS = 524288
_NW = 32
_PPW = _NPTS // _NW
_P = 128
_NCHUNK = _PPW // _P
_NG = _P // 16
_BLK = 2048
_BOUNDS = 1.6

_BCAST_DNUMS = lax.GatherDimensionNumbers(
    offset_dims=(), collapsed_slice_dims=(0,), start_index_map=(0,))


def _bcast_lane(vec, ii):
    return lax.gather(vec, ii[:, None], _BCAST_DNUMS, slice_sizes=(1,),
                      mode=lax.GatherScatterMode.PROMISE_IN_BOUNDS)


def _lane_permute(vec, idx):
    return lax.gather(vec, idx[:, None], _BCAST_DNUMS, slice_sizes=(1,),
                      mode=lax.GatherScatterMode.PROMISE_IN_BOUNDS)


def _corner_offsets(res):
    return (0, 1, res, res + 1,
            res * res, res * res + 1, res * res + res, res * res + res + 1)


def _sc_pass(res, merge_prev):
    out_cols = 2 * _OUTC if merge_prev else _OUTC
    col0 = _OUTC if merge_prev else 0
    mesh = plsc.VectorSubcoreMesh(core_axis_name="c", subcore_axis_name="s")

    scratch = [
        pltpu.VMEM((_BLK,), jnp.float32),
        pltpu.VMEM((_BLK,), jnp.float32),
        pltpu.VMEM((_BLK,), jnp.float32),
    ] + [
        pltpu.VMEM((8, _P), jnp.int32)
        for _ in range(2)
    ] + [
        pltpu.VMEM((_P, _OUTC // 2), jnp.int32)
        for _ in range(16)
    ] + [
        pltpu.VMEM((3, _P), jnp.float32)
        for _ in range(2)
    ] + [
        pltpu.VMEM((_P, out_cols), jnp.float32)
        for _ in range(2)
    ]
    if merge_prev:
        scratch += [
            pltpu.VMEM((_P, _OUTC), jnp.float32)
            for _ in range(2)
        ]
    scratch += [
        pltpu.SemaphoreType.DMA,
        pltpu.SemaphoreType.DMA,
        pltpu.SemaphoreType.DMA,
        pltpu.SemaphoreType.DMA,
    ]

    @functools.partial(
        pl.kernel,
        mesh=mesh,
        compiler_params=pltpu.CompilerParams(use_tc_tiling_on_sc=False),
        out_type=jax.ShapeDtypeStruct((_NPTS, out_cols), jnp.float32),
        scratch_types=scratch,
    )
    def sample(*args):
        if merge_prev:
            px_hbm, py_hbm, pz_hbm, t_hbm, prev_hbm, out_hbm = args[:6]
        else:
            px_hbm, py_hbm, pz_hbm, t_hbm, out_hbm = args[:5]
            prev_hbm = None
        refs = args[6:] if merge_prev else args[5:]
        pxb, pyb, pzb = refs[0:3]
        idxbs = refs[3:5]
        cbufs = (refs[5:13], refs[13:21])
        wbufs = refs[21:23]
        outbs = refs[23:25]
        refs = refs[25:]
        if merge_prev:
            fbufs = refs[0:2]
            refs = refs[2:]
        gsems = refs[0:2]
        osems = refs[2:4]

        cid = lax.axis_index("c")
        sid = lax.axis_index("s")
        wid = sid * 2 + cid
        base0 = wid * _PPW

        def load_block(blk):
            off = base0 + blk * _BLK
            pltpu.sync_copy(px_hbm.at[pl.ds(off, _BLK)], pxb)
            pltpu.sync_copy(py_hbm.at[pl.ds(off, _BLK)], pyb)
            pltpu.sync_copy(pz_hbm.at[pl.ds(off, _BLK)], pzb)

        def compute_idx(cc, slot):
            idxb = idxbs[slot]
            wbuf = wbufs[slot]
            local = (cc & (_BLK // _P - 1)) * _P

            def idx_body(g, c2):
                s = pl.ds(local + g * 16, 16)
                so = pl.ds(g * 16, 16)
                vx = pxb[s]
                vy = pyb[s]
                vz = pzb[s]
                a = jnp.float32(-0.5 * (res - 1) / _BOUNDS)
                b = jnp.float32(0.5 * (res - 1))
                hi = jnp.float32(res - 1)
                x = jnp.clip(vx * a + b, 0.0, hi)
                y = jnp.clip(vy * a + b, 0.0, hi)
                z = jnp.clip(vz * a + b, 0.0, hi)
                x0 = jnp.minimum(x.astype(jnp.int32), res - 2)
                y0 = jnp.minimum(y.astype(jnp.int32), res - 2)
                z0 = jnp.minimum(z.astype(jnp.int32), res - 2)
                wbuf[0, so] = x - x0.astype(jnp.float32)
                wbuf[1, so] = y - y0.astype(jnp.float32)
                wbuf[2, so] = z - z0.astype(jnp.float32)
                row = (z0 * res + y0) * res + x0
                for j, off in enumerate(_corner_offsets(res)):
                    idxb[j, so] = row + off
                return c2

            lax.fori_loop(0, _NG, idx_body, 0)

        def gather_copies(cc, slot):
            idxb = idxbs[slot]
            cps = []
            for j in range(8):
                cps.append(pltpu.make_async_copy(
                    t_hbm.at[idxb.at[j]], cbufs[slot][j], gsems[slot]))
            if merge_prev:
                cps.append(pltpu.make_async_copy(
                    prev_hbm.at[pl.ds(base0 + cc * _P, _P)], fbufs[slot],
                    gsems[slot]))
            return cps

        def fire(cc, slot):
            for cp in gather_copies(cc, slot):
                cp.start()

        def drain(cc, slot):
            for cp in gather_copies(cc, slot):
                cp.wait()

        def out_copy(cc, slot):
            return pltpu.make_async_copy(
                outbs[slot], out_hbm.at[pl.ds(base0 + cc * _P, _P)],
                osems[slot])

        def combine(cc, slot):
            wbuf = wbufs[slot]
            outb = outbs[slot]

            def comb_body(g, c2):
                s = pl.ds(g * 16, 16)
                wx = wbuf[0, s]
                wy = wbuf[1, s]
                wz = wbuf[2, s]
                ux = 1.0 - wx
                uy = 1.0 - wy
                uz = 1.0 - wz
                zy00 = uz * uy
                zy01 = uz * wy
                zy10 = wz * uy
                zy11 = wz * wy
                wg = (zy00 * ux, zy00 * wx, zy01 * ux, zy01 * wx,
                      zy10 * ux, zy10 * wx, zy11 * ux, zy11 * wx)
                for i in range(16):
                    p = g * 16 + i
                    ii = jnp.full((16,), i, jnp.int32)
                    bw = [_bcast_lane(wg[j], ii) for j in range(8)]
                    lo = None
                    hi = None
                    for j in range(8):
                        cb = cbufs[slot][j]
                        vi = cb[p, pl.ds(0, 16)]
                        lof = lax.bitcast_convert_type(
                            lax.shift_left(vi, 16), jnp.float32)
                        hif = lax.bitcast_convert_type(
                            jnp.bitwise_and(vi, jnp.int32(-65536)),
                            jnp.float32)
                        if j == 0:
                            lo = bw[0] * lof
                            hi = bw[0] * hif
                        else:
                            lo = lo + bw[j] * lof
                            hi = hi + bw[j] * hif
                    lane = lax.iota(jnp.int32, 16)
                    even = (lane & 1) == 0
                    idx_a = lax.shift_right_logical(lane, 1)
                    idx_b = idx_a + 8
                    half0 = jnp.where(even, _lane_permute(lo, idx_a),
                                      _lane_permute(hi, idx_a))
                    half1 = jnp.where(even, _lane_permute(lo, idx_b),
                                      _lane_permute(hi, idx_b))
                    outb[p, pl.ds(col0, 16)] = half0
                    outb[p, pl.ds(col0 + 16, 16)] = half1
                    if merge_prev:
                        fb = fbufs[slot]
                        outb[p, pl.ds(0, 16)] = fb[p, pl.ds(0, 16)]
                        outb[p, pl.ds(16, 16)] = fb[p, pl.ds(16, 16)]
                return c2

            lax.fori_loop(0, _NG, comb_body, 0)

        load_block(0)
        compute_idx(0, 0)
        fire(0, 0)

        def loop_body(i, carry):
            for u in range(2):
                cc = 2 * i + u
                par = u
                nxt = cc + 1
                npar = 1 - u
                if u == 1:
                    @pl.when(jnp.logical_and(nxt < _NCHUNK,
                                             (nxt & (_BLK // _P - 1)) == 0))
                    def _():
                        load_block(nxt // (_BLK // _P))

                @pl.when(nxt < _NCHUNK)
                def _():
                    compute_idx(nxt, npar)
                    fire(nxt, npar)

                drain(cc, par)

                @pl.when(cc >= 2)
                def _():
                    out_copy(cc - 2, par).wait()

                combine(cc, par)
                out_copy(cc, par).start()
            return carry

        lax.fori_loop(0, _NCHUNK // 2, loop_body, 0)
        out_copy(_NCHUNK - 2, 0).wait()
        out_copy(_NCHUNK - 1, 1).wait()

    return sample


def _pack_table(vol, res):
    nv = res ** 3
    a = vol.reshape(_OUTC, nv // 128, 128)
    b = jnp.transpose(a, (1, 0, 2))
    c = jnp.transpose(b, (0, 2, 1)).astype(jnp.bfloat16)
    return lax.bitcast_convert_type(
        c.reshape(nv, _OUTC // 2, 2), jnp.int32)


def kernel(pts, grid, grid2):
    t1 = _pack_table(grid[0], _R1)
    t2 = _pack_table(grid2[0], _R2)
    px = pts[:, 0]
    py = pts[:, 1]
    pz = pts[:, 2]
    feat1 = _sc_pass(_R1, False)(px, py, pz, t1)
    return _sc_pass(_R2, True)(px, py, pz, t2, feat1)


# speedup vs baseline: 1.5717x; 1.5717x over previous
import functools

import jax
import jax.numpy as jnp
from jax import lax
from jax.experimental import pallas as pl
from jax.experimental.pallas import tpu as pltpu
from jax.experimental.pallas import tpu_sc as plsc

_OUTC = 32
_R1 = 64
_R2 = 128
_NPTS = 524288
_NW = 32
_PPW = _NPTS // _NW
_P = 128
_NCHUNK = _PPW // _P
_NG = _P // 16
_BLK = 2048
_BOUNDS = 1.6

_BCAST_DNUMS = lax.GatherDimensionNumbers(
    offset_dims=(), collapsed_slice_dims=(0,), start_index_map=(0,))


def _bcast_lane(vec, ii):
    return lax.gather(vec, ii[:, None], _BCAST_DNUMS, slice_sizes=(1,),
                      mode=lax.GatherScatterMode.PROMISE_IN_BOUNDS)


def _lane_permute(vec, idx):
    return lax.gather(vec, idx[:, None], _BCAST_DNUMS, slice_sizes=(1,),
                      mode=lax.GatherScatterMode.PROMISE_IN_BOUNDS)


def _corner_offsets(res):
    return (0, 1, res, res + 1,
            res * res, res * res + 1, res * res + res, res * res + res + 1)


def _sc_pass(res, merge_prev):
    out_cols = 2 * _OUTC if merge_prev else _OUTC
    col0 = _OUTC if merge_prev else 0
    mesh = plsc.VectorSubcoreMesh(core_axis_name="c", subcore_axis_name="s")

    scratch = [
        pltpu.VMEM((_BLK,), jnp.float32),
        pltpu.VMEM((_BLK,), jnp.float32),
        pltpu.VMEM((_BLK,), jnp.float32),
    ] + [
        pltpu.VMEM((8, _P), jnp.int32)
        for _ in range(2)
    ] + [
        pltpu.VMEM((_P, _OUTC), jnp.float32)
        for _ in range(16)
    ] + [
        pltpu.VMEM((3, _P), jnp.float32)
        for _ in range(2)
    ] + [
        pltpu.VMEM((_P, out_cols), jnp.float32)
        for _ in range(2)
    ]
    if merge_prev:
        scratch += [
            pltpu.VMEM((_P, _OUTC), jnp.float32)
            for _ in range(2)
        ]
    scratch += [
        pltpu.SemaphoreType.DMA,
        pltpu.SemaphoreType.DMA,
        pltpu.SemaphoreType.DMA,
        pltpu.SemaphoreType.DMA,
    ]

    @functools.partial(
        pl.kernel,
        mesh=mesh,
        compiler_params=pltpu.CompilerParams(use_tc_tiling_on_sc=False),
        out_type=jax.ShapeDtypeStruct((_NPTS, out_cols), jnp.float32),
        scratch_types=scratch,
    )
    def sample(*args):
        if merge_prev:
            px_hbm, py_hbm, pz_hbm, t_hbm, prev_hbm, out_hbm = args[:6]
        else:
            px_hbm, py_hbm, pz_hbm, t_hbm, out_hbm = args[:5]
            prev_hbm = None
        refs = args[6:] if merge_prev else args[5:]
        pxb, pyb, pzb = refs[0:3]
        idxbs = refs[3:5]
        cbufs = (refs[5:13], refs[13:21])
        wbufs = refs[21:23]
        outbs = refs[23:25]
        refs = refs[25:]
        if merge_prev:
            fbufs = refs[0:2]
            refs = refs[2:]
        gsems = refs[0:2]
        osems = refs[2:4]

        cid = lax.axis_index("c")
        sid = lax.axis_index("s")
        wid = sid * 2 + cid
        base0 = wid * _PPW

        def load_block(blk):
            off = base0 + blk * _BLK
            pltpu.sync_copy(px_hbm.at[pl.ds(off, _BLK)], pxb)
            pltpu.sync_copy(py_hbm.at[pl.ds(off, _BLK)], pyb)
            pltpu.sync_copy(pz_hbm.at[pl.ds(off, _BLK)], pzb)

        def compute_idx(cc, slot):
            idxb = idxbs[slot]
            wbuf = wbufs[slot]
            local = (cc & (_BLK // _P - 1)) * _P

            def idx_body(g, c2):
                s = pl.ds(local + g * 16, 16)
                so = pl.ds(g * 16, 16)
                vx = pxb[s]
                vy = pyb[s]
                vz = pzb[s]
                a = jnp.float32(-0.5 * (res - 1) / _BOUNDS)
                b = jnp.float32(0.5 * (res - 1))
                hi = jnp.float32(res - 1)
                x = jnp.clip(vx * a + b, 0.0, hi)
                y = jnp.clip(vy * a + b, 0.0, hi)
                z = jnp.clip(vz * a + b, 0.0, hi)
                x0 = jnp.minimum(x.astype(jnp.int32), res - 2)
                y0 = jnp.minimum(y.astype(jnp.int32), res - 2)
                z0 = jnp.minimum(z.astype(jnp.int32), res - 2)
                wbuf[0, so] = x - x0.astype(jnp.float32)
                wbuf[1, so] = y - y0.astype(jnp.float32)
                wbuf[2, so] = z - z0.astype(jnp.float32)
                row = (z0 * res + y0) * res + x0
                for j, off in enumerate(_corner_offsets(res)):
                    idxb[j, so] = row + off
                return c2

            lax.fori_loop(0, _NG, idx_body, 0)

        def gather_copies(cc, slot):
            idxb = idxbs[slot]
            cps = []
            for j in range(8):
                cps.append(pltpu.make_async_copy(
                    t_hbm.at[idxb.at[j]], cbufs[slot][j], gsems[slot]))
            if merge_prev:
                cps.append(pltpu.make_async_copy(
                    prev_hbm.at[pl.ds(base0 + cc * _P, _P)], fbufs[slot],
                    gsems[slot]))
            return cps

        def fire(cc, slot):
            for cp in gather_copies(cc, slot):
                cp.start()

        def drain(cc, slot):
            for cp in gather_copies(cc, slot):
                cp.wait()

        def out_copy(cc, slot):
            return pltpu.make_async_copy(
                outbs[slot], out_hbm.at[pl.ds(base0 + cc * _P, _P)],
                osems[slot])

        def combine(cc, slot):
            wbuf = wbufs[slot]
            outb = outbs[slot]

            def comb_body(g, c2):
                s = pl.ds(g * 16, 16)
                wx = wbuf[0, s]
                wy = wbuf[1, s]
                wz = wbuf[2, s]
                ux = 1.0 - wx
                uy = 1.0 - wy
                uz = 1.0 - wz
                zy00 = uz * uy
                zy01 = uz * wy
                zy10 = wz * uy
                zy11 = wz * wy
                wg = (zy00 * ux, zy00 * wx, zy01 * ux, zy01 * wx,
                      zy10 * ux, zy10 * wx, zy11 * ux, zy11 * wx)
                for i in range(16):
                    p = g * 16 + i
                    ii = jnp.full((16,), i, jnp.int32)
                    bw = [_bcast_lane(wg[j], ii) for j in range(8)]
                    cb = cbufs[slot][0]
                    lo = bw[0] * cb[p, pl.ds(0, 16)]
                    hi = bw[0] * cb[p, pl.ds(16, 16)]
                    for j in range(1, 8):
                        cb = cbufs[slot][j]
                        lo = lo + bw[j] * cb[p, pl.ds(0, 16)]
                        hi = hi + bw[j] * cb[p, pl.ds(16, 16)]
                    outb[p, pl.ds(col0, 16)] = lo
                    outb[p, pl.ds(col0 + 16, 16)] = hi
                    if merge_prev:
                        fb = fbufs[slot]
                        outb[p, pl.ds(0, 16)] = fb[p, pl.ds(0, 16)]
                        outb[p, pl.ds(16, 16)] = fb[p, pl.ds(16, 16)]
                return c2

            lax.fori_loop(0, _NG, comb_body, 0)

        load_block(0)
        compute_idx(0, 0)
        fire(0, 0)

        def loop_body(i, carry):
            for u in range(2):
                cc = 2 * i + u
                par = u
                nxt = cc + 1
                npar = 1 - u
                if u == 1:
                    @pl.when(jnp.logical_and(nxt < _NCHUNK,
                                             (nxt & (_BLK // _P - 1)) == 0))
                    def _():
                        load_block(nxt // (_BLK // _P))

                @pl.when(nxt < _NCHUNK)
                def _():
                    compute_idx(nxt, npar)
                    fire(nxt, npar)

                drain(cc, par)

                @pl.when(cc >= 2)
                def _():
                    out_copy(cc - 2, par).wait()

                combine(cc, par)
                out_copy(cc, par).start()
            return carry

        lax.fori_loop(0, _NCHUNK // 2, loop_body, 0)
        out_copy(_NCHUNK - 2, 0).wait()
        out_copy(_NCHUNK - 1, 1).wait()

    return sample


def _make_table(vol, res):
    return jnp.transpose(vol, (1, 2, 3, 0)).reshape(res ** 3, _OUTC)


def kernel(pts, grid, grid2):
    t1 = _make_table(grid[0], _R1)
    t2 = _make_table(grid2[0], _R2)
    px = pts[:, 0]
    py = pts[:, 1]
    pz = pts[:, 2]
    feat1 = _sc_pass(_R1, False)(px, py, pz, t1)
    return _sc_pass(_R2, True)(px, py, pz, t2, feat1)


# speedup vs baseline: 1.5725x; 1.0005x over previous
import functools

import jax
import jax.numpy as jnp
from jax import lax
from jax.experimental import pallas as pl
from jax.experimental.pallas import tpu as pltpu
from jax.experimental.pallas import tpu_sc as plsc

_OUTC = 32
_R1 = 64
_R2 = 128
_NPTS = 524288
_NW = 32
_PPW = _NPTS // _NW
_P = 128
_NCHUNK = _PPW // _P
_NG = _P // 16
_BLK = 2048
_BOUNDS = 1.6

_BCAST_DNUMS = lax.GatherDimensionNumbers(
    offset_dims=(), collapsed_slice_dims=(0,), start_index_map=(0,))


def _bcast_lane(vec, ii):
    return lax.gather(vec, ii[:, None], _BCAST_DNUMS, slice_sizes=(1,),
                      mode=lax.GatherScatterMode.PROMISE_IN_BOUNDS)


def _lane_permute(vec, idx):
    return lax.gather(vec, idx[:, None], _BCAST_DNUMS, slice_sizes=(1,),
                      mode=lax.GatherScatterMode.PROMISE_IN_BOUNDS)


def _corner_offsets(res):
    return (0, 1, res, res + 1,
            res * res, res * res + 1, res * res + res, res * res + res + 1)


def _sc_pass(res, merge_prev):
    out_cols = 2 * _OUTC if merge_prev else _OUTC
    col0 = _OUTC if merge_prev else 0
    mesh = plsc.VectorSubcoreMesh(core_axis_name="c", subcore_axis_name="s")

    scratch = [
        pltpu.VMEM((_BLK,), jnp.float32),
        pltpu.VMEM((_BLK,), jnp.float32),
        pltpu.VMEM((_BLK,), jnp.float32),
    ] + [
        pltpu.VMEM((8, _P), jnp.int32)
        for _ in range(2)
    ] + [
        pltpu.VMEM((_P, _OUTC), jnp.float32)
        for _ in range(16)
    ] + [
        pltpu.VMEM((3, _P), jnp.float32)
        for _ in range(2)
    ] + [
        (pltpu.VMEM((_P // 2, 128), jnp.float32) if merge_prev
         else pltpu.VMEM((_P, out_cols), jnp.float32))
        for _ in range(2)
    ]
    if merge_prev:
        scratch += [
            pltpu.VMEM((_P, _OUTC), jnp.float32)
            for _ in range(2)
        ]
    scratch += [
        pltpu.SemaphoreType.DMA,
        pltpu.SemaphoreType.DMA,
        pltpu.SemaphoreType.DMA,
        pltpu.SemaphoreType.DMA,
    ]

    @functools.partial(
        pl.kernel,
        mesh=mesh,
        compiler_params=pltpu.CompilerParams(use_tc_tiling_on_sc=False),
        out_type=jax.ShapeDtypeStruct(
            (_NPTS // 2, 128) if merge_prev else (_NPTS, out_cols),
            jnp.float32),
        scratch_types=scratch,
    )
    def sample(*args):
        if merge_prev:
            px_hbm, py_hbm, pz_hbm, t_hbm, prev_hbm, out_hbm = args[:6]
        else:
            px_hbm, py_hbm, pz_hbm, t_hbm, out_hbm = args[:5]
            prev_hbm = None
        refs = args[6:] if merge_prev else args[5:]
        pxb, pyb, pzb = refs[0:3]
        idxbs = refs[3:5]
        cbufs = (refs[5:13], refs[13:21])
        wbufs = refs[21:23]
        outbs = refs[23:25]
        refs = refs[25:]
        if merge_prev:
            fbufs = refs[0:2]
            refs = refs[2:]
        gsems = refs[0:2]
        osems = refs[2:4]

        cid = lax.axis_index("c")
        sid = lax.axis_index("s")
        wid = sid * 2 + cid
        base0 = wid * _PPW

        def load_block(blk):
            off = base0 + blk * _BLK
            pltpu.sync_copy(px_hbm.at[pl.ds(off, _BLK)], pxb)
            pltpu.sync_copy(py_hbm.at[pl.ds(off, _BLK)], pyb)
            pltpu.sync_copy(pz_hbm.at[pl.ds(off, _BLK)], pzb)

        def compute_idx(cc, slot):
            idxb = idxbs[slot]
            wbuf = wbufs[slot]
            local = (cc & (_BLK // _P - 1)) * _P

            def idx_body(g, c2):
                s = pl.ds(local + g * 16, 16)
                so = pl.ds(g * 16, 16)
                vx = pxb[s]
                vy = pyb[s]
                vz = pzb[s]
                a = jnp.float32(-0.5 * (res - 1) / _BOUNDS)
                b = jnp.float32(0.5 * (res - 1))
                hi = jnp.float32(res - 1)
                x = jnp.clip(vx * a + b, 0.0, hi)
                y = jnp.clip(vy * a + b, 0.0, hi)
                z = jnp.clip(vz * a + b, 0.0, hi)
                x0 = jnp.minimum(x.astype(jnp.int32), res - 2)
                y0 = jnp.minimum(y.astype(jnp.int32), res - 2)
                z0 = jnp.minimum(z.astype(jnp.int32), res - 2)
                wbuf[0, so] = x - x0.astype(jnp.float32)
                wbuf[1, so] = y - y0.astype(jnp.float32)
                wbuf[2, so] = z - z0.astype(jnp.float32)
                row = (z0 * res + y0) * res + x0
                for j, off in enumerate(_corner_offsets(res)):
                    idxb[j, so] = row + off
                return c2

            lax.fori_loop(0, _NG, idx_body, 0)

        def gather_copies(cc, slot):
            idxb = idxbs[slot]
            cps = []
            for j in range(8):
                cps.append(pltpu.make_async_copy(
                    t_hbm.at[idxb.at[j]], cbufs[slot][j], gsems[slot]))
            if merge_prev:
                cps.append(pltpu.make_async_copy(
                    prev_hbm.at[pl.ds(base0 + cc * _P, _P)], fbufs[slot],
                    gsems[slot]))
            return cps

        def fire(cc, slot):
            for cp in gather_copies(cc, slot):
                cp.start()

        def drain(cc, slot):
            for cp in gather_copies(cc, slot):
                cp.wait()

        def out_copy(cc, slot):
            if merge_prev:
                dst = out_hbm.at[pl.ds((base0 + cc * _P) // 2, _P // 2)]
            else:
                dst = out_hbm.at[pl.ds(base0 + cc * _P, _P)]
            return pltpu.make_async_copy(outbs[slot], dst, osems[slot])

        def combine(cc, slot):
            wbuf = wbufs[slot]
            outb = outbs[slot]

            def comb_body(g, c2):
                s = pl.ds(g * 16, 16)
                wx = wbuf[0, s]
                wy = wbuf[1, s]
                wz = wbuf[2, s]
                ux = 1.0 - wx
                uy = 1.0 - wy
                uz = 1.0 - wz
                zy00 = uz * uy
                zy01 = uz * wy
                zy10 = wz * uy
                zy11 = wz * wy
                wg = (zy00 * ux, zy00 * wx, zy01 * ux, zy01 * wx,
                      zy10 * ux, zy10 * wx, zy11 * ux, zy11 * wx)
                for i in range(16):
                    p = g * 16 + i
                    ii = jnp.full((16,), i, jnp.int32)
                    bw = [_bcast_lane(wg[j], ii) for j in range(8)]
                    cb = cbufs[slot][0]
                    lo = bw[0] * cb[p, pl.ds(0, 16)]
                    hi = bw[0] * cb[p, pl.ds(16, 16)]
                    for j in range(1, 8):
                        cb = cbufs[slot][j]
                        lo = lo + bw[j] * cb[p, pl.ds(0, 16)]
                        hi = hi + bw[j] * cb[p, pl.ds(16, 16)]
                    if merge_prev:
                        pr = p >> 1
                        pc = (p & 1) * 64
                        outb[pr, pl.ds(pc + col0, 16)] = lo
                        outb[pr, pl.ds(pc + col0 + 16, 16)] = hi
                        fb = fbufs[slot]
                        outb[pr, pl.ds(pc, 16)] = fb[p, pl.ds(0, 16)]
                        outb[pr, pl.ds(pc + 16, 16)] = fb[p, pl.ds(16, 16)]
                    else:
                        outb[p, pl.ds(col0, 16)] = lo
                        outb[p, pl.ds(col0 + 16, 16)] = hi
                return c2

            lax.fori_loop(0, _NG, comb_body, 0)

        load_block(0)
        compute_idx(0, 0)
        fire(0, 0)

        def loop_body(i, carry):
            for u in range(2):
                cc = 2 * i + u
                par = u
                nxt = cc + 1
                npar = 1 - u
                if u == 1:
                    @pl.when(jnp.logical_and(nxt < _NCHUNK,
                                             (nxt & (_BLK // _P - 1)) == 0))
                    def _():
                        load_block(nxt // (_BLK // _P))

                @pl.when(nxt < _NCHUNK)
                def _():
                    compute_idx(nxt, npar)
                    fire(nxt, npar)

                drain(cc, par)

                @pl.when(cc >= 2)
                def _():
                    out_copy(cc - 2, par).wait()

                combine(cc, par)
                out_copy(cc, par).start()
            return carry

        lax.fori_loop(0, _NCHUNK // 2, loop_body, 0)
        out_copy(_NCHUNK - 2, 0).wait()
        out_copy(_NCHUNK - 1, 1).wait()

    return sample


def _make_table(vol, res):
    return jnp.transpose(vol, (1, 2, 3, 0)).reshape(res ** 3, _OUTC)


def kernel(pts, grid, grid2):
    t1 = _make_table(grid[0], _R1)
    t2 = _make_table(grid2[0], _R2)
    px = pts[:, 0]
    py = pts[:, 1]
    pz = pts[:, 2]
    feat1 = _sc_pass(_R1, False)(px, py, pz, t1)
    packed = _sc_pass(_R2, True)(px, py, pz, t2, feat1)
    return packed.reshape(_NPTS, 2 * _OUTC)
